# Initial kernel scaffold; baseline (speedup 1.0000x reference)
import jax
import jax.numpy as jnp
from jax.experimental import pallas as pl


def kernel(before_pseudoimages, after_pseudoimages, points, voxel_coords, W1, b1, W2, b2):
    raise NotImplementedError("write your pallas kernel here")


# speedup vs baseline: 5.8856x; 5.8856x over previous
import functools

import jax
import jax.numpy as jnp
from jax import lax
from jax.experimental import pallas as pl
from jax.experimental.pallas import tpu as pltpu
from jax.experimental.pallas import tpu_sc as plsc

_LANES = 16
_CHUNK = 128
_T = 2048


def _transform_body(c, before_ref, after_ref, w1_ref, b1_ref, w2_ref, b2_ref, f_ref):
    x = before_ref[0]
    y = after_ref[0]
    w1 = w1_ref[...]
    w2 = w2_ref[...]
    a = jnp.dot(w1[:c], w2, preferred_element_type=jnp.float32)
    bm = jnp.dot(w1[c:], w2, preferred_element_type=jnp.float32)
    bias = jnp.dot(b1_ref[...], w2, preferred_element_type=jnp.float32) + b2_ref[...]
    out = lax.dot_general(x, a, (((0,), (0,)), ((), ())),
                          preferred_element_type=jnp.float32)
    out = out + lax.dot_general(y, bm, (((0,), (0,)), ((), ())),
                                preferred_element_type=jnp.float32)
    f_ref[0] = out + bias[None, :]


def _transform(before2, after2, w1, b1, w2p, b2p):
    b, c, hw = before2.shape
    grid = (b, hw // _T)
    return pl.pallas_call(
        functools.partial(_transform_body, c),
        grid=grid,
        in_specs=[
            pl.BlockSpec((1, c, _T), lambda i, j: (i, 0, j)),
            pl.BlockSpec((1, c, _T), lambda i, j: (i, 0, j)),
            pl.BlockSpec((2 * c, 32), lambda i, j: (0, 0)),
            pl.BlockSpec((32,), lambda i, j: (0,)),
            pl.BlockSpec((32, _LANES), lambda i, j: (0, 0)),
            pl.BlockSpec((_LANES,), lambda i, j: (0,)),
        ],
        out_specs=pl.BlockSpec((1, _T, _LANES), lambda i, j: (i, j, 0)),
        out_shape=jax.ShapeDtypeStruct((b, hw, _LANES), jnp.float32),
    )(before2, after2, w1, b1, w2p, b2p)


def _make_gather(total_rows, total_pad, n_per_batch, hw, w, nbatch):
    info = plsc.get_sparse_core_info()
    nw = info.num_cores * info.num_subcores
    cpw = total_pad // nw
    nchunk = cpw // _CHUNK
    mesh = plsc.VectorSubcoreMesh(core_axis_name="c", subcore_axis_name="s")

    @functools.partial(
        pl.kernel,
        mesh=mesh,
        compiler_params=pltpu.CompilerParams(use_tc_tiling_on_sc=False),
        out_type=jax.ShapeDtypeStruct((total_pad, _LANES), jnp.float32),
        scratch_types=[
            pltpu.VMEM((cpw,), jnp.int32),
            pltpu.VMEM((cpw,), jnp.int32),
            pltpu.VMEM((nchunk, _CHUNK), jnp.int32),
            pltpu.VMEM((_CHUNK, _LANES), jnp.float32),
            pltpu.VMEM((_CHUNK, _LANES), jnp.float32),
            pltpu.SemaphoreType.DMA,
            pltpu.SemaphoreType.DMA,
        ],
    )
    def gather_kernel(f_hbm, y_hbm, x_hbm, out_hbm, y_v, x_v, idx_v, r0, r1, s0, s1):
        wid = lax.axis_index("s") * info.num_cores + lax.axis_index("c")
        base = wid * cpw
        pltpu.sync_copy(y_hbm.at[pl.ds(base, cpw)], y_v)
        pltpu.sync_copy(x_hbm.at[pl.ds(base, cpw)], x_v)

        lanes = lax.iota(jnp.int32, 16)

        def chunk_idx_body(j, _):
            def lane_body(g, _):
                off = j * _CHUNK + g * 16
                pos = base + off + lanes
                yy = y_v[pl.ds(off, 16)]
                xx = x_v[pl.ds(off, 16)]
                bidx = lax.div(pos, jnp.full((16,), n_per_batch, jnp.int32))
                bidx = lax.min(bidx, jnp.full((16,), nbatch - 1, jnp.int32))
                idx_v[j, pl.ds(g * 16, 16)] = bidx * hw + yy * w + xx
                return 0
            lax.fori_loop(0, _CHUNK // 16, lane_body, 0)
            return 0

        lax.fori_loop(0, nchunk, chunk_idx_body, 0)

        def seq_body(j, _):
            pltpu.async_copy(f_hbm.at[idx_v.at[j]], r0, s0).wait()
            pltpu.sync_copy(r0, out_hbm.at[pl.ds(base + j * _CHUNK, _CHUNK)])
            return 0

        lax.fori_loop(0, nchunk, seq_body, 0)

    return gather_kernel


def kernel(before_pseudoimages, after_pseudoimages, points, voxel_coords, W1, b1, W2, b2):
    b, c, h, w = before_pseudoimages.shape
    n = voxel_coords.shape[1]
    hw = h * w

    w2p = jnp.zeros((W2.shape[0], _LANES), jnp.float32).at[:, : W2.shape[1]].set(W2)
    b2p = jnp.zeros((_LANES,), jnp.float32).at[: b2.shape[0]].set(b2)

    f = _transform(
        before_pseudoimages.reshape(b, c, hw),
        after_pseudoimages.reshape(b, c, hw),
        W1, b1, w2p, b2p,
    )
    f_flat = f.reshape(b * hw, _LANES)

    total = b * n
    nw = 32
    cpw = -(-total // (nw * _CHUNK)) * _CHUNK
    total_pad = cpw * nw

    yf = voxel_coords[:, :, 1].reshape(-1).astype(jnp.int32)
    xf = voxel_coords[:, :, 2].reshape(-1).astype(jnp.int32)
    yf = jnp.pad(yf, (0, total_pad - total))
    xf = jnp.pad(xf, (0, total_pad - total))

    gathered = _make_gather(total, total_pad, n, hw, w, b)(f_flat, yf, xf)
    return gathered[:total, :3].reshape(b, n, 3)


# speedup vs baseline: 6.6259x; 1.1258x over previous
import functools

import jax
import jax.numpy as jnp
from jax import lax
from jax.experimental import pallas as pl
from jax.experimental.pallas import tpu as pltpu
from jax.experimental.pallas import tpu_sc as plsc

_LANES = 16
_CHUNK = 128
_T = 8192


def _transform_body(c, before_ref, after_ref, w1_ref, b1_ref, w2_ref, b2_ref, f_ref):
    x = before_ref[0]
    y = after_ref[0]
    w1 = w1_ref[...]
    w2 = w2_ref[...]
    a = jnp.dot(w1[:c], w2, preferred_element_type=jnp.float32)
    bm = jnp.dot(w1[c:], w2, preferred_element_type=jnp.float32)
    bias = jnp.dot(b1_ref[...], w2, preferred_element_type=jnp.float32) + b2_ref[...]
    out = lax.dot_general(x, a, (((0,), (0,)), ((), ())),
                          preferred_element_type=jnp.float32)
    out = out + lax.dot_general(y, bm, (((0,), (0,)), ((), ())),
                                preferred_element_type=jnp.float32)
    f_ref[0] = out + bias[None, :]


def _transform(before2, after2, w1, b1, w2p, b2p):
    b, c, hw = before2.shape
    grid = (b, hw // _T)
    return pl.pallas_call(
        functools.partial(_transform_body, c),
        grid=grid,
        in_specs=[
            pl.BlockSpec((1, c, _T), lambda i, j: (i, 0, j)),
            pl.BlockSpec((1, c, _T), lambda i, j: (i, 0, j)),
            pl.BlockSpec((2 * c, 32), lambda i, j: (0, 0)),
            pl.BlockSpec((32,), lambda i, j: (0,)),
            pl.BlockSpec((32, _LANES), lambda i, j: (0, 0)),
            pl.BlockSpec((_LANES,), lambda i, j: (0,)),
        ],
        out_specs=pl.BlockSpec((1, _T, _LANES), lambda i, j: (i, j, 0)),
        out_shape=jax.ShapeDtypeStruct((b, hw, _LANES), jnp.float32),
    )(before2, after2, w1, b1, w2p, b2p)


def _make_gather(total_rows, total_pad, n_per_batch, hw, w, nbatch):
    info = plsc.get_sparse_core_info()
    nw = info.num_cores * info.num_subcores
    cpw = total_pad // nw
    nchunk = cpw // _CHUNK
    mesh = plsc.VectorSubcoreMesh(core_axis_name="c", subcore_axis_name="s")

    @functools.partial(
        pl.kernel,
        mesh=mesh,
        compiler_params=pltpu.CompilerParams(use_tc_tiling_on_sc=False),
        out_type=jax.ShapeDtypeStruct((total_pad, _LANES), jnp.float32),
        scratch_types=[
            pltpu.VMEM((cpw,), jnp.int32),
            pltpu.VMEM((cpw,), jnp.int32),
            pltpu.VMEM((nchunk, _CHUNK), jnp.int32),
            pltpu.VMEM((_CHUNK, _LANES), jnp.float32),
            pltpu.VMEM((_CHUNK, _LANES), jnp.float32),
            pltpu.SemaphoreType.DMA,
            pltpu.SemaphoreType.DMA,
        ],
    )
    def gather_kernel(f_hbm, y_hbm, x_hbm, out_hbm, y_v, x_v, idx_v, r0, r1, s0, s1):
        wid = lax.axis_index("s") * info.num_cores + lax.axis_index("c")
        base = wid * cpw
        pltpu.sync_copy(y_hbm.at[pl.ds(base, cpw)], y_v)
        pltpu.sync_copy(x_hbm.at[pl.ds(base, cpw)], x_v)

        lanes = lax.iota(jnp.int32, 16)

        def chunk_idx_body(j, _):
            def lane_body(g, _):
                off = j * _CHUNK + g * 16
                pos = base + off + lanes
                yy = y_v[pl.ds(off, 16)]
                xx = x_v[pl.ds(off, 16)]
                bidx = lax.div(pos, jnp.full((16,), n_per_batch, jnp.int32))
                bidx = lax.min(bidx, jnp.full((16,), nbatch - 1, jnp.int32))
                idx_v[j, pl.ds(g * 16, 16)] = bidx * hw + yy * w + xx
                return 0
            lax.fori_loop(0, _CHUNK // 16, lane_body, 0)
            return 0

        lax.fori_loop(0, nchunk, chunk_idx_body, 0)

        def seq_body(j, _):
            pltpu.async_copy(f_hbm.at[idx_v.at[j]], r0, s0).wait()
            pltpu.sync_copy(r0, out_hbm.at[pl.ds(base + j * _CHUNK, _CHUNK)])
            return 0

        lax.fori_loop(0, nchunk, seq_body, 0)

    return gather_kernel


def kernel(before_pseudoimages, after_pseudoimages, points, voxel_coords, W1, b1, W2, b2):
    b, c, h, w = before_pseudoimages.shape
    n = voxel_coords.shape[1]
    hw = h * w

    w2p = jnp.zeros((W2.shape[0], _LANES), jnp.float32).at[:, : W2.shape[1]].set(W2)
    b2p = jnp.zeros((_LANES,), jnp.float32).at[: b2.shape[0]].set(b2)

    f = _transform(
        before_pseudoimages.reshape(b, c, hw),
        after_pseudoimages.reshape(b, c, hw),
        W1, b1, w2p, b2p,
    )
    f_flat = f.reshape(b * hw, _LANES)

    total = b * n
    nw = 32
    cpw = -(-total // (nw * _CHUNK)) * _CHUNK
    total_pad = cpw * nw

    yf = voxel_coords[:, :, 1].reshape(-1).astype(jnp.int32)
    xf = voxel_coords[:, :, 2].reshape(-1).astype(jnp.int32)
    yf = jnp.pad(yf, (0, total_pad - total))
    xf = jnp.pad(xf, (0, total_pad - total))

    gathered = _make_gather(total, total_pad, n, hw, w, b)(f_flat, yf, xf)
    return gathered[:total, :3].reshape(b, n, 3)
